# Initial kernel scaffold; baseline (speedup 1.0000x reference)
import jax
import jax.numpy as jnp
from jax.experimental import pallas as pl


def kernel(x, W, b):
    raise NotImplementedError("write your pallas kernel here")


# speedup vs baseline: 3.5030x; 3.5030x over previous
import functools

import jax
import jax.numpy as jnp
from jax.experimental import pallas as pl

N_EXPERTS = 64
TOPK = 8
BLOCK_ROWS = 256


def _router_block(x_ref, wt_ref, b_ref, out_ref, idx_ref):
    logits = (
        jnp.dot(x_ref[:], wt_ref[:], preferred_element_type=jnp.float32)
        + b_ref[:]
    )
    br = logits.shape[0]
    lane = jax.lax.broadcasted_iota(jnp.int32, (br, N_EXPERTS), 1)
    vals = logits
    selected = jnp.zeros((br, N_EXPERTS), dtype=jnp.bool_)
    idx_cols = []
    row_max = None
    for k in range(TOPK):
        m = jnp.max(vals, axis=-1, keepdims=True)
        if k == 0:
            row_max = m
        idx = jnp.min(
            jnp.where(vals == m, lane, N_EXPERTS), axis=-1, keepdims=True
        )
        idx_cols.append(idx)
        onehot = lane == idx
        selected = jnp.logical_or(selected, onehot)
        vals = jnp.where(onehot, -jnp.inf, vals)
    idx_ref[:] = jnp.concatenate(idx_cols, axis=1)
    e = jnp.where(selected, jnp.exp(logits - row_max), 0.0)
    out_ref[:] = e / jnp.sum(e, axis=-1, keepdims=True)


@jax.jit
def kernel(x, W, b):
    n_tokens = x.shape[0]
    d_model = x.shape[1]
    wt = W.T
    b2 = b.reshape(1, N_EXPERTS)
    grid = (n_tokens // BLOCK_ROWS,)
    out, idx = pl.pallas_call(
        _router_block,
        grid=grid,
        in_specs=[
            pl.BlockSpec((BLOCK_ROWS, d_model), lambda i: (i, 0)),
            pl.BlockSpec((d_model, N_EXPERTS), lambda i: (0, 0)),
            pl.BlockSpec((1, N_EXPERTS), lambda i: (0, 0)),
        ],
        out_specs=[
            pl.BlockSpec((BLOCK_ROWS, N_EXPERTS), lambda i: (i, 0)),
            pl.BlockSpec((BLOCK_ROWS, TOPK), lambda i: (i, 0)),
        ],
        out_shape=[
            jax.ShapeDtypeStruct((n_tokens, N_EXPERTS), jnp.float32),
            jax.ShapeDtypeStruct((n_tokens, TOPK), jnp.int32),
        ],
    )(x, wt, b2)
    return (out, idx)


# speedup vs baseline: 4.2240x; 1.2058x over previous
import functools

import jax
import jax.numpy as jnp
from jax.experimental import pallas as pl

N_EXPERTS = 64
TOPK = 8
BLOCK_ROWS = 256


def _router_block(x_ref, wt_ref, b_ref, out_ref, idx_ref):
    logits = (
        jnp.dot(x_ref[:], wt_ref[:], preferred_element_type=jnp.float32)
        + b_ref[:]
    )
    br = logits.shape[0]
    lanef = jax.lax.broadcasted_iota(jnp.int32, (br, N_EXPERTS), 1).astype(
        jnp.float32
    )
    vals = logits
    idx_cols = []
    row_max = None
    for k in range(TOPK):
        m = jnp.max(vals, axis=-1, keepdims=True)
        if k == 0:
            row_max = m
        idxf = jnp.min(
            jnp.where(vals == m, lanef, jnp.float32(N_EXPERTS)),
            axis=-1,
            keepdims=True,
        )
        idx_cols.append(idxf)
        vals = jnp.where(lanef == idxf, -jnp.inf, vals)
    idx_ref[:] = jnp.concatenate(idx_cols, axis=1).astype(jnp.int32)
    e = jnp.where(vals == -jnp.inf, jnp.exp(logits - row_max), 0.0)
    out_ref[:] = e / jnp.sum(e, axis=-1, keepdims=True)


@jax.jit
def kernel(x, W, b):
    n_tokens = x.shape[0]
    d_model = x.shape[1]
    wt = W.T
    b2 = b.reshape(1, N_EXPERTS)
    grid = (n_tokens // BLOCK_ROWS,)
    out, idx = pl.pallas_call(
        _router_block,
        grid=grid,
        in_specs=[
            pl.BlockSpec((BLOCK_ROWS, d_model), lambda i: (i, 0)),
            pl.BlockSpec((d_model, N_EXPERTS), lambda i: (0, 0)),
            pl.BlockSpec((1, N_EXPERTS), lambda i: (0, 0)),
        ],
        out_specs=[
            pl.BlockSpec((BLOCK_ROWS, N_EXPERTS), lambda i: (i, 0)),
            pl.BlockSpec((BLOCK_ROWS, TOPK), lambda i: (i, 0)),
        ],
        out_shape=[
            jax.ShapeDtypeStruct((n_tokens, N_EXPERTS), jnp.float32),
            jax.ShapeDtypeStruct((n_tokens, TOPK), jnp.int32),
        ],
    )(x, wt, b2)
    return (out, idx)


# speedup vs baseline: 5.9854x; 1.4170x over previous
import functools

import jax
import jax.numpy as jnp
from jax.experimental import pallas as pl

N_EXPERTS = 64
TOPK = 8
BLOCK_TOK = 256


def _router_block(w_ref, b_ref, x_ref, out_ref, idx_ref):
    logits = (
        jax.lax.dot_general(
            w_ref[:],
            x_ref[:],
            (((1,), (1,)), ((), ())),
            preferred_element_type=jnp.float32,
        )
        + b_ref[:]
    )
    bt = logits.shape[1]
    lane = jax.lax.broadcasted_iota(jnp.int32, (N_EXPERTS, bt), 0).astype(
        jnp.float32
    )
    vals = logits
    idx_rows = []
    row_max = None
    for k in range(TOPK):
        m = jnp.max(vals, axis=0, keepdims=True)
        if k == 0:
            row_max = m
        idxf = jnp.min(
            jnp.where(vals == m, lane, jnp.float32(N_EXPERTS)),
            axis=0,
            keepdims=True,
        )
        idx_rows.append(idxf)
        vals = jnp.where(lane == idxf, -jnp.inf, vals)
    idx_ref[:] = jnp.concatenate(idx_rows, axis=0).astype(jnp.int32)
    e = jnp.where(vals == -jnp.inf, jnp.exp(logits - row_max), 0.0)
    out_ref[:] = e / jnp.sum(e, axis=0, keepdims=True)


@jax.jit
def kernel(x, W, b):
    n_tokens = x.shape[0]
    d_model = x.shape[1]
    b2 = b.reshape(N_EXPERTS, 1)
    grid = (n_tokens // BLOCK_TOK,)
    outT, idxT = pl.pallas_call(
        _router_block,
        grid=grid,
        in_specs=[
            pl.BlockSpec((N_EXPERTS, d_model), lambda i: (0, 0)),
            pl.BlockSpec((N_EXPERTS, 1), lambda i: (0, 0)),
            pl.BlockSpec((BLOCK_TOK, d_model), lambda i: (i, 0)),
        ],
        out_specs=[
            pl.BlockSpec((N_EXPERTS, BLOCK_TOK), lambda i: (0, i)),
            pl.BlockSpec((TOPK, BLOCK_TOK), lambda i: (0, i)),
        ],
        out_shape=[
            jax.ShapeDtypeStruct((N_EXPERTS, n_tokens), jnp.float32),
            jax.ShapeDtypeStruct((TOPK, n_tokens), jnp.int32),
        ],
    )(W, b2, x)
    return (outT.T, idxT.T)


# speedup vs baseline: 6.2634x; 1.0465x over previous
import functools

import jax
import jax.numpy as jnp
from jax.experimental import pallas as pl

N_EXPERTS = 64
TOPK = 8
BLOCK_TOK = 512


def _router_block(w_ref, b_ref, x_ref, out_ref, idx_ref):
    logits = (
        jax.lax.dot_general(
            w_ref[:],
            x_ref[:],
            (((1,), (1,)), ((), ())),
            preferred_element_type=jnp.float32,
        )
        + b_ref[:]
    )
    bt = logits.shape[1]
    lane = jax.lax.broadcasted_iota(jnp.int32, (N_EXPERTS, bt), 0).astype(
        jnp.float32
    )
    vals = logits
    idx_rows = []
    row_max = None
    for k in range(TOPK):
        m = jnp.max(vals, axis=0, keepdims=True)
        if k == 0:
            row_max = m
        idxf = jnp.min(
            jnp.where(vals == m, lane, jnp.float32(N_EXPERTS)),
            axis=0,
            keepdims=True,
        )
        idx_rows.append(idxf)
        vals = jnp.where(lane == idxf, -jnp.inf, vals)
    idxT = jnp.concatenate(idx_rows, axis=0).astype(jnp.int32)
    idx_ref[:] = idxT.T
    e = jnp.where(vals == -jnp.inf, jnp.exp(logits - row_max), 0.0)
    out_ref[:] = (e / jnp.sum(e, axis=0, keepdims=True)).T


@jax.jit
def kernel(x, W, b):
    n_tokens = x.shape[0]
    d_model = x.shape[1]
    b2 = b.reshape(N_EXPERTS, 1)
    grid = (n_tokens // BLOCK_TOK,)
    out, idx = pl.pallas_call(
        _router_block,
        grid=grid,
        in_specs=[
            pl.BlockSpec((N_EXPERTS, d_model), lambda i: (0, 0)),
            pl.BlockSpec((N_EXPERTS, 1), lambda i: (0, 0)),
            pl.BlockSpec((BLOCK_TOK, d_model), lambda i: (i, 0)),
        ],
        out_specs=[
            pl.BlockSpec((BLOCK_TOK, N_EXPERTS), lambda i: (i, 0)),
            pl.BlockSpec((BLOCK_TOK, TOPK), lambda i: (i, 0)),
        ],
        out_shape=[
            jax.ShapeDtypeStruct((n_tokens, N_EXPERTS), jnp.float32),
            jax.ShapeDtypeStruct((n_tokens, TOPK), jnp.int32),
        ],
    )(W, b2, x)
    return (out, idx)


# speedup vs baseline: 6.7573x; 1.0788x over previous
import functools

import jax
import jax.numpy as jnp
from jax.experimental import pallas as pl

N_EXPERTS = 64
TOPK = 8
BLOCK_TOK = 1024


def _router_block(w_ref, b_ref, x_ref, out_ref, idx_ref):
    logits = (
        jax.lax.dot_general(
            w_ref[:],
            x_ref[:],
            (((1,), (1,)), ((), ())),
            preferred_element_type=jnp.float32,
        )
        + b_ref[:]
    )
    bt = logits.shape[1]
    lane = jax.lax.broadcasted_iota(jnp.int32, (N_EXPERTS, bt), 0).astype(
        jnp.float32
    )
    vals = logits
    idx_rows = []
    row_max = None
    for k in range(TOPK):
        m = jnp.max(vals, axis=0, keepdims=True)
        if k == 0:
            row_max = m
        idxf = jnp.min(
            jnp.where(vals == m, lane, jnp.float32(N_EXPERTS)),
            axis=0,
            keepdims=True,
        )
        idx_rows.append(idxf)
        vals = jnp.where(lane == idxf, -jnp.inf, vals)
    idxT = jnp.concatenate(idx_rows, axis=0).astype(jnp.int32)
    idx_ref[:] = idxT.T
    e = jnp.where(vals == -jnp.inf, jnp.exp(logits - row_max), 0.0)
    out_ref[:] = (e / jnp.sum(e, axis=0, keepdims=True)).T


@jax.jit
def kernel(x, W, b):
    n_tokens = x.shape[0]
    d_model = x.shape[1]
    b2 = b.reshape(N_EXPERTS, 1)
    grid = (n_tokens // BLOCK_TOK,)
    out, idx = pl.pallas_call(
        _router_block,
        grid=grid,
        in_specs=[
            pl.BlockSpec((N_EXPERTS, d_model), lambda i: (0, 0)),
            pl.BlockSpec((N_EXPERTS, 1), lambda i: (0, 0)),
            pl.BlockSpec((BLOCK_TOK, d_model), lambda i: (i, 0)),
        ],
        out_specs=[
            pl.BlockSpec((BLOCK_TOK, N_EXPERTS), lambda i: (i, 0)),
            pl.BlockSpec((BLOCK_TOK, TOPK), lambda i: (i, 0)),
        ],
        out_shape=[
            jax.ShapeDtypeStruct((n_tokens, N_EXPERTS), jnp.float32),
            jax.ShapeDtypeStruct((n_tokens, TOPK), jnp.int32),
        ],
    )(W, b2, x)
    return (out, idx)
